# baseline (speedup 1.0000x reference)
import numpy as np

import jax
import jax.numpy as jnp
from jax.experimental import pallas as pl
from jax.experimental.pallas import tpu as pltpu

_VMEM_LIMIT = 48 * 1024 * 1024


def _lrelu(y):
    return jnp.where(y >= 0.0, y, 0.2 * y)


def _col_gather_mat(w_in):
    w2 = w_in // 2
    g = np.zeros((w_in, 4 * w2), np.float32)
    for v in range(4):
        for j in range(w2):
            c = 2 * j + v - 1
            if 0 <= c < w_in:
                g[c, v * w2 + j] = 1.0
    return jnp.asarray(g, dtype=jnp.bfloat16)


def _lo16(v):
    return jax.lax.bitcast_convert_type(v.astype(jnp.int16), jnp.bfloat16)


def _hi16(v):
    return jax.lax.bitcast_convert_type(
        jax.lax.shift_right_logical(v, 16).astype(jnp.int16), jnp.bfloat16)


def _tap_planes(a3, gmat):
    c_in, h_in, w_in = a3.shape
    h2, lanes = h_in // 2, 4 * (w_in // 2)
    a2 = a3.reshape(c_in * h_in, w_in)
    yc = jnp.dot(a2, gmat, preferred_element_type=jnp.float32)
    yi = pltpu.bitcast(yc.astype(jnp.bfloat16), jnp.int32)
    yi = yi.reshape(c_in, h2, lanes)
    zr = jnp.zeros((c_in, 1, lanes), jnp.int32)
    yi_dn = jnp.concatenate([zr, yi[:, :h2 - 1, :]], axis=1)
    yi_up = jnp.concatenate([yi[:, 1:, :], zr], axis=1)
    return _hi16(yi_dn), _lo16(yi), _hi16(yi), _lo16(yi_up)


def _norm_act(y, instance_norm):
    if instance_norm:
        m = jnp.mean(y, axis=(1, 2), keepdims=True)
        var = jnp.maximum(
            jnp.mean(y * y, axis=(1, 2), keepdims=True) - m * m, 0.0)
        y = (y - m) * jax.lax.rsqrt(var + 1e-5)
    return _lrelu(y).astype(jnp.bfloat16)


def _conv1_s2(a3, wmat, gmat, bias):
    c_in, h_in, w_in = a3.shape
    w2 = w_in // 2
    p_kh = _tap_planes(a3, gmat)
    pieces = [p_kh[kh][:, :, v * w2:(v + 1) * w2]
              for kh in range(4) for v in range(4)]
    patches = jnp.concatenate(pieces, axis=0)
    y = jnp.einsum("cf,fij->cij", wmat, patches,
                   preferred_element_type=jnp.float32) + bias
    return _norm_act(y, False)


def _conv_s2(a3, wexp, gmat):
    c_in, h_in, w_in = a3.shape
    w2 = w_in // 2
    cout = wexp.shape[0] // 4
    pk = jnp.concatenate(_tap_planes(a3, gmat), axis=0)
    yb = jnp.einsum("cf,fij->cij", wexp, pk,
                    preferred_element_type=jnp.float32)
    y = yb[0:cout, :, 0:w2]
    for v in range(1, 4):
        y = y + yb[v * cout:(v + 1) * cout, :, v * w2:(v + 1) * w2]
    return _norm_act(y, True)


def _stack_kernel(x_ref, w1_ref, b1_ref, g1_ref, w2_ref, g2_ref, w3_ref,
                  g3_ref, w4_ref, g4_ref, w5_ref, g5_ref, o_ref):
    for s in range(1):
        xb = x_ref[s].astype(jnp.bfloat16)
        a = _conv1_s2(xb, w1_ref[...], g1_ref[...], b1_ref[...])
        a = _conv_s2(a, w2_ref[...], g2_ref[...])
        a = _conv_s2(a, w3_ref[...], g3_ref[...])
        a = _conv_s2(a, w4_ref[...], g4_ref[...])
        a = _conv_s2(a, w5_ref[...], g5_ref[...])
        o_ref[s] = a


def _head_kernel(z_ref, w1_ref, b1_ref, w2_ref, b2_ref, w3_ref, b3_ref,
                 o_ref):
    h = jnp.dot(z_ref[...], w1_ref[...],
                preferred_element_type=jnp.float32) + b1_ref[...]
    h = _lrelu(h).astype(jnp.bfloat16)
    h = jnp.dot(h, w2_ref[...],
                preferred_element_type=jnp.float32) + b2_ref[...]
    h = _lrelu(h)
    logit = jnp.sum(h * w3_ref[...].astype(jnp.float32),
                    axis=-1, keepdims=True) + b3_ref[...]
    e = jnp.exp(-jnp.abs(logit))
    o_ref[...] = jnp.where(logit >= 0.0, 1.0 / (1.0 + e), e / (1.0 + e))


def _full_spec(shape):
    return pl.BlockSpec(shape, lambda i: (0,) * len(shape))


def kernel(x, c1_w, c1_b, c2_w, c2_b, c3_w, c3_b, c4_w, c4_b, c5_w, c5_b,
           fc1_w, fc1_b, fc2_w, fc2_b, fc3_w, fc3_b):
    del c2_b, c3_b, c4_b, c5_b
    batch = x.shape[0]
    w1 = c1_w.reshape(2, 2, 4, 2, 2, 5)
    w1 = jnp.transpose(w1, (2, 0, 3, 1, 4, 5)).reshape(4, 80)
    b1 = c1_b.reshape(4, 1, 1)
    gmats = [_col_gather_mat(w_in) for w_in in (256, 128, 64, 32, 16)]

    def expand_w(wmat):
        cout = wmat.shape[0]
        c = wmat.shape[1] // 16
        w4 = wmat.reshape(cout, 4, 4, c)
        return jnp.transpose(w4, (2, 0, 1, 3)).reshape(4 * cout, 4 * c)

    w2e, w3e, w4e, w5e = (expand_w(w) for w in (c2_w, c3_w, c4_w, c5_w))

    feat = pl.pallas_call(
        _stack_kernel,
        out_shape=jax.ShapeDtypeStruct((batch, 16, 8, 8), jnp.bfloat16),
        grid=(batch,),
        in_specs=[
            pl.BlockSpec((1, 5, 256, 256), lambda i: (i, 0, 0, 0)),
            _full_spec((4, 80)),
            _full_spec((4, 1, 1)),
            _full_spec((256, 512)),
            _full_spec((32, 16)),
            _full_spec((128, 256)),
            _full_spec((64, 32)),
            _full_spec((64, 128)),
            _full_spec((128, 64)),
            _full_spec((32, 64)),
            _full_spec((64, 128)),
            _full_spec((16, 32)),
        ],
        out_specs=pl.BlockSpec((1, 16, 8, 8), lambda i: (i, 0, 0, 0)),
        compiler_params=pltpu.CompilerParams(
            dimension_semantics=("arbitrary",),
            vmem_limit_bytes=_VMEM_LIMIT),
    )(x, w1, b1, gmats[0], w2e, gmats[1], w3e, gmats[2], w4e, gmats[3],
      w5e, gmats[4])

    z = feat.reshape(batch, 16 * 8 * 8)
    return pl.pallas_call(
        _head_kernel,
        out_shape=jax.ShapeDtypeStruct((batch, 1), jnp.float32),
        compiler_params=pltpu.CompilerParams(vmem_limit_bytes=_VMEM_LIMIT),
    )(z, fc1_w, fc1_b, fc2_w, fc2_b, fc3_w, fc3_b)


# speedup vs baseline: 1.0545x; 1.0545x over previous
import numpy as np

import jax
import jax.numpy as jnp
from jax.experimental import pallas as pl
from jax.experimental.pallas import tpu as pltpu

_VMEM_LIMIT = 48 * 1024 * 1024


def _lrelu(y):
    return jnp.where(y >= 0.0, y, 0.2 * y)


def _col_gather_mat(w_in):
    w2 = w_in // 2
    g = np.zeros((w_in, 4 * w2), np.float32)
    for v in range(4):
        for j in range(w2):
            c = 2 * j + v - 1
            if 0 <= c < w_in:
                g[c, v * w2 + j] = 1.0
    return jnp.asarray(g, dtype=jnp.bfloat16)


def _lo16(v):
    return jax.lax.bitcast_convert_type(v.astype(jnp.int16), jnp.bfloat16)


def _hi16(v):
    return jax.lax.bitcast_convert_type(
        jax.lax.shift_right_logical(v, 16).astype(jnp.int16), jnp.bfloat16)


def _tap_planes(a3, gmat):
    c_in, h_in, w_in = a3.shape
    h2, lanes = h_in // 2, 4 * (w_in // 2)
    a2 = a3.reshape(c_in * h_in, w_in)
    yc = jnp.dot(a2, gmat, preferred_element_type=jnp.float32)
    yi = pltpu.bitcast(yc.astype(jnp.bfloat16), jnp.int32)
    yi = yi.reshape(c_in, h2, lanes)
    zr = jnp.zeros((c_in, 1, lanes), jnp.int32)
    yi_dn = jnp.concatenate([zr, yi[:, :h2 - 1, :]], axis=1)
    yi_up = jnp.concatenate([yi[:, 1:, :], zr], axis=1)
    return _hi16(yi_dn), _lo16(yi), _hi16(yi), _lo16(yi_up)


def _norm_act(y, instance_norm):
    if instance_norm:
        m = jnp.mean(y, axis=(1, 2), keepdims=True)
        var = jnp.maximum(
            jnp.mean(y * y, axis=(1, 2), keepdims=True) - m * m, 0.0)
        y = (y - m) * jax.lax.rsqrt(var + 1e-5)
    return _lrelu(y).astype(jnp.bfloat16)


def _conv1_s2(a3, wmat, gmat, bias):
    c_in, h_in, w_in = a3.shape
    w2 = w_in // 2
    p_kh = _tap_planes(a3, gmat)
    pieces = [p_kh[kh][:, :, v * w2:(v + 1) * w2]
              for kh in range(4) for v in range(4)]
    patches = jnp.concatenate(pieces, axis=0)
    y = jnp.einsum("cf,fij->cij", wmat, patches,
                   preferred_element_type=jnp.float32) + bias
    return _norm_act(y, False)


def _conv_s2(a3, wexp, gmat):
    c_in, h_in, w_in = a3.shape
    w2 = w_in // 2
    cout = wexp.shape[0] // 4
    pk = jnp.concatenate(_tap_planes(a3, gmat), axis=0)
    yb = jnp.einsum("cf,fij->cij", wexp, pk,
                    preferred_element_type=jnp.float32)
    y = yb[0:cout, :, 0:w2]
    for v in range(1, 4):
        y = y + yb[v * cout:(v + 1) * cout, :, v * w2:(v + 1) * w2]
    return _norm_act(y, True)


def _stack_kernel(x_ref, w1_ref, b1_ref, g1_ref, w2_ref, g2_ref, w3_ref,
                  g3_ref, w4_ref, g4_ref, w5_ref, g5_ref, o_ref):
    for s in range(2):
        xb = x_ref[s].astype(jnp.bfloat16)
        a = _conv1_s2(xb, w1_ref[...], g1_ref[...], b1_ref[...])
        a = _conv_s2(a, w2_ref[...], g2_ref[...])
        a = _conv_s2(a, w3_ref[...], g3_ref[...])
        a = _conv_s2(a, w4_ref[...], g4_ref[...])
        a = _conv_s2(a, w5_ref[...], g5_ref[...])
        o_ref[s] = a


def _head_kernel(z_ref, w1_ref, b1_ref, w2_ref, b2_ref, w3_ref, b3_ref,
                 o_ref):
    h = jnp.dot(z_ref[...], w1_ref[...],
                preferred_element_type=jnp.float32) + b1_ref[...]
    h = _lrelu(h).astype(jnp.bfloat16)
    h = jnp.dot(h, w2_ref[...],
                preferred_element_type=jnp.float32) + b2_ref[...]
    h = _lrelu(h)
    logit = jnp.sum(h * w3_ref[...].astype(jnp.float32),
                    axis=-1, keepdims=True) + b3_ref[...]
    e = jnp.exp(-jnp.abs(logit))
    o_ref[...] = jnp.where(logit >= 0.0, 1.0 / (1.0 + e), e / (1.0 + e))


def _full_spec(shape):
    return pl.BlockSpec(shape, lambda i: (0,) * len(shape))


def kernel(x, c1_w, c1_b, c2_w, c2_b, c3_w, c3_b, c4_w, c4_b, c5_w, c5_b,
           fc1_w, fc1_b, fc2_w, fc2_b, fc3_w, fc3_b):
    del c2_b, c3_b, c4_b, c5_b
    batch = x.shape[0]
    w1 = c1_w.reshape(2, 2, 4, 2, 2, 5)
    w1 = jnp.transpose(w1, (2, 0, 3, 1, 4, 5)).reshape(4, 80)
    b1 = c1_b.reshape(4, 1, 1)
    gmats = [_col_gather_mat(w_in) for w_in in (256, 128, 64, 32, 16)]

    def expand_w(wmat):
        cout = wmat.shape[0]
        c = wmat.shape[1] // 16
        w4 = wmat.reshape(cout, 4, 4, c)
        return jnp.transpose(w4, (2, 0, 1, 3)).reshape(4 * cout, 4 * c)

    w2e, w3e, w4e, w5e = (expand_w(w) for w in (c2_w, c3_w, c4_w, c5_w))

    feat = pl.pallas_call(
        _stack_kernel,
        out_shape=jax.ShapeDtypeStruct((batch, 16, 8, 8), jnp.bfloat16),
        grid=(batch // 2,),
        in_specs=[
            pl.BlockSpec((2, 5, 256, 256), lambda i: (i, 0, 0, 0)),
            _full_spec((4, 80)),
            _full_spec((4, 1, 1)),
            _full_spec((256, 512)),
            _full_spec((32, 16)),
            _full_spec((128, 256)),
            _full_spec((64, 32)),
            _full_spec((64, 128)),
            _full_spec((128, 64)),
            _full_spec((32, 64)),
            _full_spec((64, 128)),
            _full_spec((16, 32)),
        ],
        out_specs=pl.BlockSpec((2, 16, 8, 8), lambda i: (i, 0, 0, 0)),
        compiler_params=pltpu.CompilerParams(
            dimension_semantics=("arbitrary",),
            vmem_limit_bytes=_VMEM_LIMIT),
    )(x, w1, b1, gmats[0], w2e, gmats[1], w3e, gmats[2], w4e, gmats[3],
      w5e, gmats[4])

    z = feat.reshape(batch, 16 * 8 * 8)
    return pl.pallas_call(
        _head_kernel,
        out_shape=jax.ShapeDtypeStruct((batch, 1), jnp.float32),
        compiler_params=pltpu.CompilerParams(vmem_limit_bytes=_VMEM_LIMIT),
    )(z, fc1_w, fc1_b, fc2_w, fc2_b, fc3_w, fc3_b)
